# baseline (speedup 1.0000x reference)
import functools

import jax
import jax.numpy as jnp
from jax import lax
from jax.experimental import pallas as pl
from jax.experimental.pallas import tpu as pltpu
from jax.experimental.pallas import tpu_sc as plsc

N = 10000
E = 320000
H = 128
HC = H // 2
OUT = 64

NC = 2
NS = 16
NW = NC * NS
CHUNK = 256
BANKC = 8
NB = 10
CPW = NB * BANKC
NG = CPW
E_PAD = CPW * NS * CHUNK

EMB_CHUNK = 128
EMB_CPT = -(-N // (NS * EMB_CHUNK))
N_PAD = EMB_CPT * NS * EMB_CHUNK
AGG_ROWS = N + 16
AGG_PER_TILE = AGG_ROWS // NS
DEG_LEN = 10112
DEG_PER_TILE = DEG_LEN // NS


def _mesh():
    return plsc.VectorSubcoreMesh(
        core_axis_name="c", subcore_axis_name="s", num_cores=NC, num_subcores=NS
    )


_SC_PARAMS = pltpu.CompilerParams(use_tc_tiling_on_sc=False)


def _sc_l0_body(key_hbm, val_hbm, kidx_h, vidx_h, src_h, dst_h,
                zagg_h, zdeg_h, ones_h,
                h0c_h, aggp_h, degp_h,
                kidx_v, vidx_v, kbuf, vbuf, hbuf,
                sidxb, didxb, ones_v, ebufs, agg_sh, deg_sh,
                sem_e, sem_h, sem_i, sem_g, sem_s, sem_d):
    c = lax.axis_index("c")
    s = lax.axis_index("s")

    zbase = s * AGG_PER_TILE
    pltpu.sync_copy(zagg_h.at[pl.ds(zbase, AGG_PER_TILE)],
                    agg_sh.at[pl.ds(zbase, AGG_PER_TILE)])
    dzbase = s * DEG_PER_TILE
    pltpu.sync_copy(zdeg_h.at[pl.ds(dzbase, DEG_PER_TILE)],
                    deg_sh.at[pl.ds(dzbase, DEG_PER_TILE)])
    pltpu.sync_copy(ones_h, ones_v)

    pltpu.sync_copy(kidx_h.at[pl.ds(s * EMB_CPT, EMB_CPT)], kidx_v)
    pltpu.sync_copy(vidx_h.at[pl.ds(s * EMB_CPT, EMB_CPT)], vidx_v)
    col0 = c * HC

    for t in range(EMB_CPT):
        kg = pltpu.async_copy(key_hbm.at[kidx_v.at[t]], kbuf, sem_e)
        vg = pltpu.async_copy(val_hbm.at[vidx_v.at[t]], vbuf, sem_e)
        kg.wait()
        vg.wait()

        def row_body(r, carry):
            for j in range(HC // 16):
                off = col0 + j * 16
                k16 = kbuf[r, pl.ds(off, 16)]
                v16 = vbuf[r, pl.ds(off, 16)]
                hbuf[r, pl.ds(j * 16, 16)] = jnp.maximum(k16 + v16, 0.0)
            return carry

        lax.fori_loop(0, EMB_CHUNK, row_body, 0)
        base = (s * EMB_CPT + t) * EMB_CHUNK
        pltpu.sync_copy(hbuf, h0c_h.at[c, pl.ds(base, EMB_CHUNK)])
    plsc.subcore_barrier()

    yc = h0c_h.at[c]

    def fire_idx(b):
        slot = lax.rem(b, 2)
        pltpu.async_copy(src_h.at[pl.ds(s * CPW + b * BANKC, BANKC)],
                         sidxb.at[slot], sem_i)
        pltpu.async_copy(dst_h.at[pl.ds(s * CPW + b * BANKC, BANKC)],
                         didxb.at[slot], sem_i)

    def drain_idx():
        pltpu.make_async_copy(src_h.at[pl.ds(0, BANKC)], sidxb.at[0],
                              sem_i).wait()
        pltpu.make_async_copy(dst_h.at[pl.ds(0, BANKC)], didxb.at[0],
                              sem_i).wait()

    def fire_gather(slot, k, bank):
        pltpu.async_copy(yc.at[sidxb.at[slot, k]], ebufs.at[bank], sem_g)

    def drain_buf(bank, sem):
        pltpu.make_async_copy(yc.at[pl.ds(0, CHUNK)], ebufs.at[bank],
                              sem).wait()

    fire_idx(0)
    drain_idx()
    fire_gather(0, 0, 0)

    def bank_body(b, carry):
        bslot = lax.rem(b, 2)

        for k in range(BANKC):
            g = b * BANKC + k
            p = lax.rem(g, 2)
            drain_buf(p, sem_g)

            @pl.when(g > 0)
            def _():
                drain_buf(1 - p, sem_s)

            if k == 1:
                @pl.when(b + 1 < NB)
                def _():
                    fire_idx(b + 1)

            if k == BANKC - 1:
                @pl.when(b + 1 < NB)
                def _():
                    drain_idx()
                    fire_gather(1 - bslot, 0, 1 - p)
            else:
                fire_gather(bslot, k + 1, 1 - p)

            pltpu.async_copy(ebufs.at[p], agg_sh.at[didxb.at[bslot, k]],
                             sem_s, add=True)
            pltpu.async_copy(ones_v, deg_sh.at[didxb.at[bslot, k]],
                             sem_d, add=True)
        return carry

    lax.fori_loop(0, NB, bank_body, 0)
    drain_buf(lax.rem(NG - 1, 2), sem_s)
    for _ in range(NB):
        pltpu.make_async_copy(dst_h.at[pl.ds(0, BANKC)], didxb.at[0],
                              sem_d).wait()
    plsc.subcore_barrier()

    pltpu.sync_copy(agg_sh.at[pl.ds(zbase, AGG_PER_TILE)],
                    aggp_h.at[c, pl.ds(zbase, AGG_PER_TILE)])
    pltpu.sync_copy(deg_sh.at[pl.ds(dzbase, DEG_PER_TILE)],
                    degp_h.at[c, pl.ds(dzbase, DEG_PER_TILE)])


@jax.jit
def _sc_l0(key_emb, val_emb, kidx, vidx, src2, dst2, zagg, zdeg, ones):
    kern = pl.kernel(
        _sc_l0_body,
        out_type=(
            jax.ShapeDtypeStruct((NC, N_PAD, HC), jnp.float32),
            jax.ShapeDtypeStruct((NC, AGG_ROWS, HC), jnp.float32),
            jax.ShapeDtypeStruct((NC, DEG_LEN), jnp.float32),
        ),
        mesh=_mesh(),
        scratch_types=[
            pltpu.VMEM((EMB_CPT, EMB_CHUNK), jnp.int32),
            pltpu.VMEM((EMB_CPT, EMB_CHUNK), jnp.int32),
            pltpu.VMEM((EMB_CHUNK, H), jnp.float32),
            pltpu.VMEM((EMB_CHUNK, H), jnp.float32),
            pltpu.VMEM((EMB_CHUNK, HC), jnp.float32),
            pltpu.VMEM((2, BANKC, CHUNK), jnp.int32),
            pltpu.VMEM((2, BANKC, CHUNK), jnp.int32),
            pltpu.VMEM((CHUNK,), jnp.float32),
            pltpu.VMEM((2, CHUNK, HC), jnp.float32),
            pltpu.VMEM_SHARED((AGG_ROWS, HC), jnp.float32),
            pltpu.VMEM_SHARED((DEG_LEN,), jnp.float32),
            pltpu.SemaphoreType.DMA,
            pltpu.SemaphoreType.DMA,
            pltpu.SemaphoreType.DMA,
            pltpu.SemaphoreType.DMA,
            pltpu.SemaphoreType.DMA,
            pltpu.SemaphoreType.DMA,
        ],
        compiler_params=_SC_PARAMS,
    )
    return kern(key_emb, val_emb, kidx, vidx, src2, dst2, zagg, zdeg, ones)


def _sc_spmm_body(y_hbm, src_h, dst_h, zagg_h, aggp_h,
                  sidx_v, didx_v, bufs, agg_sh, sem_g, sem_s):
    c = lax.axis_index("c")
    s = lax.axis_index("s")

    zbase = s * AGG_PER_TILE
    pltpu.sync_copy(zagg_h.at[pl.ds(zbase, AGG_PER_TILE)],
                    agg_sh.at[pl.ds(zbase, AGG_PER_TILE)])

    pltpu.sync_copy(src_h.at[pl.ds(s * CPW, CPW)], sidx_v)
    pltpu.sync_copy(dst_h.at[pl.ds(s * CPW, CPW)], didx_v)

    yc = y_hbm.at[c]

    def fire_gather(g, bank):
        pltpu.async_copy(yc.at[sidx_v.at[g]], bufs.at[bank], sem_g)

    def drain(bank, sem):
        pltpu.make_async_copy(yc.at[pl.ds(0, CHUNK)], bufs.at[bank],
                              sem).wait()

    fire_gather(0, 0)
    plsc.subcore_barrier()

    def body(g, carry):
        p = lax.rem(g, 2)
        drain(p, sem_g)

        @pl.when(g > 0)
        def _():
            drain(1 - p, sem_s)

        @pl.when(g + 1 < NG)
        def _():
            fire_gather(g + 1, 1 - p)

        pltpu.async_copy(bufs.at[p], agg_sh.at[didx_v.at[g]], sem_s, add=True)
        return carry

    lax.fori_loop(0, NG, body, 0)
    drain(lax.rem(NG - 1, 2), sem_s)
    plsc.subcore_barrier()

    pltpu.sync_copy(agg_sh.at[pl.ds(zbase, AGG_PER_TILE)],
                    aggp_h.at[c, pl.ds(zbase, AGG_PER_TILE)])


@jax.jit
def _sc_spmm(y, src2, dst2, zagg):
    kern = pl.kernel(
        _sc_spmm_body,
        out_type=jax.ShapeDtypeStruct((NC, AGG_ROWS, HC), jnp.float32),
        mesh=_mesh(),
        scratch_types=[
            pltpu.VMEM((CPW, CHUNK), jnp.int32),
            pltpu.VMEM((CPW, CHUNK), jnp.int32),
            pltpu.VMEM((2, CHUNK, HC), jnp.float32),
            pltpu.VMEM_SHARED((AGG_ROWS, HC), jnp.float32),
            pltpu.SemaphoreType.DMA,
            pltpu.SemaphoreType.DMA,
        ],
        compiler_params=_SC_PARAMS,
    )
    return kern(y, src2, dst2, zagg)


ROWS_BLK = 1000


def _tc_b_body(h0a_ref, h0b_ref, a0_ref, a1_ref, d_ref,
               ws0_ref, wn0_ref, ws1_ref, wn1_ref, s_ref, y_ref):
    scale = 1.0 / jnp.maximum(d_ref[0], 1.0)
    h0 = jnp.concatenate([h0a_ref[0], h0b_ref[0]], axis=1)
    mean = jnp.concatenate([a0_ref[0], a1_ref[0]], axis=1) * scale
    h1 = (jnp.dot(h0, ws0_ref[...], preferred_element_type=jnp.float32)
          + jnp.dot(mean, wn0_ref[...], preferred_element_type=jnp.float32))
    s_ref[...] = jnp.dot(h1, ws1_ref[...], preferred_element_type=jnp.float32)
    y = jnp.dot(h1, wn1_ref[...], preferred_element_type=jnp.float32)
    y_ref[0] = y[:, :HC]
    y_ref[1] = y[:, HC:]


@jax.jit
def _tc_b(h0c, aggp, degp, ws0, wn0, ws1, wn1):
    grid = (N // ROWS_BLK,)
    blk = pl.BlockSpec((ROWS_BLK, H), lambda i: (i, 0))
    h0a = pl.BlockSpec((1, ROWS_BLK, HC), lambda i: (0, i, 0))
    h0b = pl.BlockSpec((1, ROWS_BLK, HC), lambda i: (1, i, 0))
    a0blk = pl.BlockSpec((1, ROWS_BLK, HC), lambda i: (0, i, 0))
    a1blk = pl.BlockSpec((1, ROWS_BLK, HC), lambda i: (1, i, 0))
    dblk = pl.BlockSpec((1, ROWS_BLK, 1), lambda i: (0, i, 0))
    wblk = pl.BlockSpec((H, H), lambda i: (0, 0))
    yblk = pl.BlockSpec((NC, ROWS_BLK, HC), lambda i: (0, i, 0))
    return pl.pallas_call(
        _tc_b_body,
        grid=grid,
        in_specs=[h0a, h0b, a0blk, a1blk, dblk, wblk, wblk, wblk, wblk],
        out_specs=[blk, yblk],
        out_shape=[
            jax.ShapeDtypeStruct((N, H), jnp.float32),
            jax.ShapeDtypeStruct((NC, N, HC), jnp.float32),
        ],
    )(h0c, h0c, aggp, aggp, degp, ws0, wn0, ws1, wn1)


def _tc_d_body(s1_ref, a0_ref, a1_ref, d_ref, wc_ref, o_ref):
    scale = 1.0 / jnp.maximum(d_ref[0], 1.0)
    agg = jnp.concatenate([a0_ref[0], a1_ref[0]], axis=1)
    h = s1_ref[...] + agg * scale
    o_ref[...] = jnp.dot(h, wc_ref[...], preferred_element_type=jnp.float32)


@jax.jit
def _tc_d(s1, aggp, degp, wc):
    grid = (N // ROWS_BLK,)
    blk = pl.BlockSpec((ROWS_BLK, H), lambda i: (i, 0))
    a0blk = pl.BlockSpec((1, ROWS_BLK, HC), lambda i: (0, i, 0))
    a1blk = pl.BlockSpec((1, ROWS_BLK, HC), lambda i: (1, i, 0))
    dblk = pl.BlockSpec((1, ROWS_BLK, 1), lambda i: (0, i, 0))
    wblk = pl.BlockSpec((H, OUT), lambda i: (0, 0))
    oblk = pl.BlockSpec((ROWS_BLK, OUT), lambda i: (i, 0))
    return pl.pallas_call(
        _tc_d_body,
        grid=grid,
        in_specs=[blk, a0blk, a1blk, dblk, wblk],
        out_specs=oblk,
        out_shape=jax.ShapeDtypeStruct((N, OUT), jnp.float32),
    )(s1, aggp, aggp, degp, wc)


def kernel(feats, edge_index, key_emb, val_emb, W_self0, W_neigh0, W_self1,
           W_neigh1, W_cls):
    kidx = jnp.pad(feats[:, 0], (0, N_PAD - N)).reshape(-1, EMB_CHUNK)
    vidx = jnp.pad(feats[:, 1], (0, N_PAD - N)).reshape(-1, EMB_CHUNK)
    src2 = jnp.pad(edge_index[0], (0, E_PAD - E)).reshape(-1, CHUNK)
    dst2 = jnp.pad(edge_index[1], (0, E_PAD - E),
                   constant_values=N).reshape(-1, CHUNK)
    zdeg = jnp.zeros((DEG_LEN,), jnp.float32)
    zagg = jnp.zeros((AGG_ROWS, HC), jnp.float32)
    ones = jnp.ones((CHUNK,), jnp.float32)

    h0c, aggp0, degp = _sc_l0(key_emb, val_emb, kidx, vidx, src2, dst2,
                              zagg, zdeg, ones)
    degp3 = degp[:1].reshape(1, DEG_LEN, 1)
    s1, y1 = _tc_b(h0c, aggp0, degp3, W_self0, W_neigh0, W_self1, W_neigh1)
    aggp1 = _sc_spmm(y1, src2, dst2, zagg)
    out = _tc_d(s1, aggp1, degp3, W_cls)
    return out


# speedup vs baseline: 1.0877x; 1.0877x over previous
import functools

import jax
import jax.numpy as jnp
from jax import lax
from jax.experimental import pallas as pl
from jax.experimental.pallas import tpu as pltpu
from jax.experimental.pallas import tpu_sc as plsc

N = 10000
E = 320000
H = 128
HC = H // 2
OUT = 64

NC = 2
NS = 16
NW = NC * NS
CHUNK = 256
BANKC = 8
NB = 10
CPW = NB * BANKC
NG = CPW
E_PAD = CPW * NS * CHUNK

EMB_CHUNK = 128
EMB_CPT = -(-N // (NS * EMB_CHUNK))
N_PAD = EMB_CPT * NS * EMB_CHUNK
AGG_ROWS = N + 112
AGG_PER_TILE = AGG_ROWS // NS
DEG_LEN = 10112
DEG_PER_TILE = DEG_LEN // NS


def _mesh():
    return plsc.VectorSubcoreMesh(
        core_axis_name="c", subcore_axis_name="s", num_cores=NC, num_subcores=NS
    )


_SC_PARAMS = pltpu.CompilerParams(use_tc_tiling_on_sc=False)


def _sc_l0_body(key_hbm, val_hbm, kidx_h, vidx_h, src_h, dst_h,
                zagg_h, zdeg_h, ones_h,
                h0c_h, aggp_h, degp_h,
                kidx_v, vidx_v, kbuf, vbuf, hbuf,
                sidxb, didxb, ones_v, ebufs, agg_sh, deg_sh,
                sem_e, sem_h, sem_i, sem_g, sem_s, sem_d):
    c = lax.axis_index("c")
    s = lax.axis_index("s")

    zbase = s * AGG_PER_TILE
    pltpu.sync_copy(zagg_h.at[pl.ds(zbase, AGG_PER_TILE)],
                    agg_sh.at[pl.ds(zbase, AGG_PER_TILE)])
    dzbase = s * DEG_PER_TILE
    pltpu.sync_copy(zdeg_h.at[pl.ds(dzbase, DEG_PER_TILE)],
                    deg_sh.at[pl.ds(dzbase, DEG_PER_TILE)])
    pltpu.sync_copy(ones_h, ones_v)

    pltpu.sync_copy(kidx_h.at[pl.ds(s * EMB_CPT, EMB_CPT)], kidx_v)
    pltpu.sync_copy(vidx_h.at[pl.ds(s * EMB_CPT, EMB_CPT)], vidx_v)
    col0 = c * HC

    for t in range(EMB_CPT):
        kg = pltpu.async_copy(key_hbm.at[kidx_v.at[t]], kbuf, sem_e)
        vg = pltpu.async_copy(val_hbm.at[vidx_v.at[t]], vbuf, sem_e)
        kg.wait()
        vg.wait()

        def row_body(r, carry):
            for j in range(HC // 16):
                off = col0 + j * 16
                k16 = kbuf[r, pl.ds(off, 16)]
                v16 = vbuf[r, pl.ds(off, 16)]
                hbuf[r, pl.ds(j * 16, 16)] = jnp.maximum(k16 + v16, 0.0)
            return carry

        lax.fori_loop(0, EMB_CHUNK, row_body, 0)
        base = (s * EMB_CPT + t) * EMB_CHUNK
        pltpu.sync_copy(hbuf, h0c_h.at[c, pl.ds(base, EMB_CHUNK)])
    plsc.subcore_barrier()

    yc = h0c_h.at[c]

    def fire_idx(b):
        slot = lax.rem(b, 2)
        pltpu.async_copy(src_h.at[pl.ds(s * CPW + b * BANKC, BANKC)],
                         sidxb.at[slot], sem_i)
        pltpu.async_copy(dst_h.at[pl.ds(s * CPW + b * BANKC, BANKC)],
                         didxb.at[slot], sem_i)

    def drain_idx():
        pltpu.make_async_copy(src_h.at[pl.ds(0, BANKC)], sidxb.at[0],
                              sem_i).wait()
        pltpu.make_async_copy(dst_h.at[pl.ds(0, BANKC)], didxb.at[0],
                              sem_i).wait()

    def fire_gather(slot, k, bank):
        pltpu.async_copy(yc.at[sidxb.at[slot, k]], ebufs.at[bank], sem_g)

    def drain_buf(bank, sem):
        pltpu.make_async_copy(yc.at[pl.ds(0, CHUNK)], ebufs.at[bank],
                              sem).wait()

    fire_idx(0)
    drain_idx()
    fire_gather(0, 0, 0)

    def bank_body(b, carry):
        bslot = lax.rem(b, 2)

        for k in range(BANKC):
            g = b * BANKC + k
            p = lax.rem(g, 2)
            drain_buf(p, sem_g)

            @pl.when(g > 0)
            def _():
                drain_buf(1 - p, sem_s)

            if k == 1:
                @pl.when(b + 1 < NB)
                def _():
                    fire_idx(b + 1)

            if k == BANKC - 1:
                @pl.when(b + 1 < NB)
                def _():
                    drain_idx()
                    fire_gather(1 - bslot, 0, 1 - p)
            else:
                fire_gather(bslot, k + 1, 1 - p)

            pltpu.async_copy(ebufs.at[p], agg_sh.at[didxb.at[bslot, k]],
                             sem_s, add=True)
            pltpu.async_copy(ones_v, deg_sh.at[didxb.at[bslot, k]],
                             sem_d, add=True)
        return carry

    lax.fori_loop(0, NB, bank_body, 0)
    drain_buf(lax.rem(NG - 1, 2), sem_s)
    for _ in range(NB):
        pltpu.make_async_copy(dst_h.at[pl.ds(0, BANKC)], didxb.at[0],
                              sem_d).wait()
    plsc.subcore_barrier()

    pltpu.sync_copy(agg_sh.at[pl.ds(zbase, AGG_PER_TILE)],
                    aggp_h.at[c, pl.ds(zbase, AGG_PER_TILE)])
    pltpu.sync_copy(deg_sh.at[pl.ds(dzbase, DEG_PER_TILE)],
                    degp_h.at[c, pl.ds(dzbase, DEG_PER_TILE)])


@jax.jit
def _sc_l0(key_emb, val_emb, kidx, vidx, src2, dst2, zagg, zdeg, ones):
    kern = pl.kernel(
        _sc_l0_body,
        out_type=(
            jax.ShapeDtypeStruct((NC, N_PAD, HC), jnp.float32),
            jax.ShapeDtypeStruct((NC, AGG_ROWS, HC), jnp.float32),
            jax.ShapeDtypeStruct((NC, DEG_LEN), jnp.float32),
        ),
        mesh=_mesh(),
        scratch_types=[
            pltpu.VMEM((EMB_CPT, EMB_CHUNK), jnp.int32),
            pltpu.VMEM((EMB_CPT, EMB_CHUNK), jnp.int32),
            pltpu.VMEM((EMB_CHUNK, H), jnp.float32),
            pltpu.VMEM((EMB_CHUNK, H), jnp.float32),
            pltpu.VMEM((EMB_CHUNK, HC), jnp.float32),
            pltpu.VMEM((2, BANKC, CHUNK), jnp.int32),
            pltpu.VMEM((2, BANKC, CHUNK), jnp.int32),
            pltpu.VMEM((CHUNK,), jnp.float32),
            pltpu.VMEM((2, CHUNK, HC), jnp.float32),
            pltpu.VMEM_SHARED((AGG_ROWS, HC), jnp.float32),
            pltpu.VMEM_SHARED((DEG_LEN,), jnp.float32),
            pltpu.SemaphoreType.DMA,
            pltpu.SemaphoreType.DMA,
            pltpu.SemaphoreType.DMA,
            pltpu.SemaphoreType.DMA,
            pltpu.SemaphoreType.DMA,
            pltpu.SemaphoreType.DMA,
        ],
        compiler_params=_SC_PARAMS,
    )
    return kern(key_emb, val_emb, kidx, vidx, src2, dst2, zagg, zdeg, ones)


def _sc_spmm_body(y_hbm, src_h, dst_h, zagg_h, aggp_h,
                  sidx_v, didx_v, bufs, agg_sh, sem_g, sem_s):
    c = lax.axis_index("c")
    s = lax.axis_index("s")

    zbase = s * AGG_PER_TILE
    pltpu.sync_copy(zagg_h.at[pl.ds(zbase, AGG_PER_TILE)],
                    agg_sh.at[pl.ds(zbase, AGG_PER_TILE)])

    pltpu.sync_copy(src_h.at[pl.ds(s * CPW, CPW)], sidx_v)
    pltpu.sync_copy(dst_h.at[pl.ds(s * CPW, CPW)], didx_v)

    yc = y_hbm.at[c]

    def fire_gather(g, bank):
        pltpu.async_copy(yc.at[sidx_v.at[g]], bufs.at[bank], sem_g)

    def drain(bank, sem):
        pltpu.make_async_copy(yc.at[pl.ds(0, CHUNK)], bufs.at[bank],
                              sem).wait()

    fire_gather(0, 0)
    plsc.subcore_barrier()

    def body(g, carry):
        p = lax.rem(g, 2)
        drain(p, sem_g)

        @pl.when(g > 0)
        def _():
            drain(1 - p, sem_s)

        @pl.when(g + 1 < NG)
        def _():
            fire_gather(g + 1, 1 - p)

        pltpu.async_copy(bufs.at[p], agg_sh.at[didx_v.at[g]], sem_s, add=True)
        return carry

    lax.fori_loop(0, NG, body, 0)
    drain(lax.rem(NG - 1, 2), sem_s)
    plsc.subcore_barrier()

    pltpu.sync_copy(agg_sh.at[pl.ds(zbase, AGG_PER_TILE)],
                    aggp_h.at[c, pl.ds(zbase, AGG_PER_TILE)])


@jax.jit
def _sc_spmm(y, src2, dst2, zagg):
    kern = pl.kernel(
        _sc_spmm_body,
        out_type=jax.ShapeDtypeStruct((NC, AGG_ROWS, HC), jnp.float32),
        mesh=_mesh(),
        scratch_types=[
            pltpu.VMEM((CPW, CHUNK), jnp.int32),
            pltpu.VMEM((CPW, CHUNK), jnp.int32),
            pltpu.VMEM((2, CHUNK, HC), jnp.float32),
            pltpu.VMEM_SHARED((AGG_ROWS, HC), jnp.float32),
            pltpu.SemaphoreType.DMA,
            pltpu.SemaphoreType.DMA,
        ],
        compiler_params=_SC_PARAMS,
    )
    return kern(y, src2, dst2, zagg)


ROWS_BLK = 1000


def _tc_b_body(h0a_ref, h0b_ref, a0_ref, a1_ref, d_ref,
               ws0_ref, wn0_ref, ws1_ref, wn1_ref, s_ref, y_ref):
    scale = 1.0 / jnp.maximum(d_ref[0], 1.0)
    h0 = jnp.concatenate([h0a_ref[0], h0b_ref[0]], axis=1)
    mean = jnp.concatenate([a0_ref[0], a1_ref[0]], axis=1) * scale
    h1 = (jnp.dot(h0, ws0_ref[...], preferred_element_type=jnp.float32)
          + jnp.dot(mean, wn0_ref[...], preferred_element_type=jnp.float32))
    s_ref[...] = jnp.dot(h1, ws1_ref[...], preferred_element_type=jnp.float32)
    y = jnp.dot(h1, wn1_ref[...], preferred_element_type=jnp.float32)
    y_ref[0] = y[:, :HC]
    y_ref[1] = y[:, HC:]


@jax.jit
def _tc_b(h0c, aggp, degp, ws0, wn0, ws1, wn1):
    grid = (N // ROWS_BLK,)
    blk = pl.BlockSpec((ROWS_BLK, H), lambda i: (i, 0))
    h0a = pl.BlockSpec((1, ROWS_BLK, HC), lambda i: (0, i, 0))
    h0b = pl.BlockSpec((1, ROWS_BLK, HC), lambda i: (1, i, 0))
    a0blk = pl.BlockSpec((1, ROWS_BLK, HC), lambda i: (0, i, 0))
    a1blk = pl.BlockSpec((1, ROWS_BLK, HC), lambda i: (1, i, 0))
    dblk = pl.BlockSpec((1, ROWS_BLK, 1), lambda i: (0, i, 0))
    wblk = pl.BlockSpec((H, H), lambda i: (0, 0))
    yblk = pl.BlockSpec((NC, ROWS_BLK, HC), lambda i: (0, i, 0))
    return pl.pallas_call(
        _tc_b_body,
        grid=grid,
        in_specs=[h0a, h0b, a0blk, a1blk, dblk, wblk, wblk, wblk, wblk],
        out_specs=[blk, yblk],
        out_shape=[
            jax.ShapeDtypeStruct((N, H), jnp.float32),
            jax.ShapeDtypeStruct((NC, N, HC), jnp.float32),
        ],
    )(h0c, h0c, aggp, aggp, degp, ws0, wn0, ws1, wn1)


def _tc_d_body(s1_ref, a0_ref, a1_ref, d_ref, wc_ref, o_ref):
    scale = 1.0 / jnp.maximum(d_ref[0], 1.0)
    agg = jnp.concatenate([a0_ref[0], a1_ref[0]], axis=1)
    h = s1_ref[...] + agg * scale
    o_ref[...] = jnp.dot(h, wc_ref[...], preferred_element_type=jnp.float32)


@jax.jit
def _tc_d(s1, aggp, degp, wc):
    grid = (N // ROWS_BLK,)
    blk = pl.BlockSpec((ROWS_BLK, H), lambda i: (i, 0))
    a0blk = pl.BlockSpec((1, ROWS_BLK, HC), lambda i: (0, i, 0))
    a1blk = pl.BlockSpec((1, ROWS_BLK, HC), lambda i: (1, i, 0))
    dblk = pl.BlockSpec((1, ROWS_BLK, 1), lambda i: (0, i, 0))
    wblk = pl.BlockSpec((H, OUT), lambda i: (0, 0))
    oblk = pl.BlockSpec((ROWS_BLK, OUT), lambda i: (i, 0))
    return pl.pallas_call(
        _tc_d_body,
        grid=grid,
        in_specs=[blk, a0blk, a1blk, dblk, wblk],
        out_specs=oblk,
        out_shape=jax.ShapeDtypeStruct((N, OUT), jnp.float32),
    )(s1, aggp, aggp, degp, wc)


def kernel(feats, edge_index, key_emb, val_emb, W_self0, W_neigh0, W_self1,
           W_neigh1, W_cls):
    kidx = jnp.pad(feats[:, 0], (0, N_PAD - N)).reshape(-1, EMB_CHUNK)
    vidx = jnp.pad(feats[:, 1], (0, N_PAD - N)).reshape(-1, EMB_CHUNK)
    src2 = jnp.pad(edge_index[0], (0, E_PAD - E)).reshape(-1, CHUNK)
    pad_dst = N + jnp.arange(E_PAD - E, dtype=jnp.int32) % 112
    dst2 = jnp.concatenate([edge_index[1], pad_dst]).reshape(-1, CHUNK)
    zdeg = jnp.zeros((DEG_LEN,), jnp.float32)
    zagg = jnp.zeros((AGG_ROWS, HC), jnp.float32)
    ones = jnp.ones((CHUNK,), jnp.float32)

    h0c, aggp0, degp = _sc_l0(key_emb, val_emb, kidx, vidx, src2, dst2,
                              zagg, zdeg, ones)
    degp3 = degp[:1].reshape(1, DEG_LEN, 1)
    s1, y1 = _tc_b(h0c, aggp0, degp3, W_self0, W_neigh0, W_self1, W_neigh1)
    aggp1 = _sc_spmm(y1, src2, dst2, zagg)
    out = _tc_d(s1, aggp1, degp3, W_cls)
    return out


# speedup vs baseline: 2.0568x; 1.8910x over previous
import functools

import jax
import jax.numpy as jnp
from jax import lax
from jax.experimental import pallas as pl
from jax.experimental.pallas import tpu as pltpu
from jax.experimental.pallas import tpu_sc as plsc

N = 10000
E = 320000
H = 128
HC = H // 2
OUT = 64

NC = 2
NS = 16
NW = NC * NS
CHUNK = 256
BANKC = 8
NB = 10
CPW = NB * BANKC
NG = CPW
E_PAD = CPW * NS * CHUNK

EMB_CHUNK = 128
EMB_CPT = -(-N // (NS * EMB_CHUNK))
N_PAD = EMB_CPT * NS * EMB_CHUNK
AGG_ROWS = N + 112
AGG_PER_TILE = AGG_ROWS // NS
DEG_LEN = 10112
DEG_PER_TILE = DEG_LEN // NS


def _mesh():
    return plsc.VectorSubcoreMesh(
        core_axis_name="c", subcore_axis_name="s", num_cores=NC, num_subcores=NS
    )


_SC_PARAMS = pltpu.CompilerParams(use_tc_tiling_on_sc=False)


def _sc_l0_body(key_hbm, val_hbm, kidx_h, vidx_h, src_h, dst_h,
                zagg_h, zdeg_h, ones_h,
                h0c_h, aggp_h, degp_h,
                kidx_v, vidx_v, kbuf, vbuf, hbuf,
                sidxb, didxb, ones_v, ebufs, agg_sh, deg_sh,
                sem_e, sem_h, sem_i, sem_g, sem_s, sem_d):
    c = lax.axis_index("c")
    s = lax.axis_index("s")

    zbase = s * AGG_PER_TILE
    pltpu.sync_copy(zagg_h.at[pl.ds(zbase, AGG_PER_TILE)],
                    agg_sh.at[pl.ds(zbase, AGG_PER_TILE)])
    dzbase = s * DEG_PER_TILE
    pltpu.sync_copy(zdeg_h.at[pl.ds(dzbase, DEG_PER_TILE)],
                    deg_sh.at[pl.ds(dzbase, DEG_PER_TILE)])
    pltpu.sync_copy(ones_h, ones_v)

    pltpu.sync_copy(kidx_h.at[pl.ds(s * EMB_CPT, EMB_CPT)], kidx_v)
    pltpu.sync_copy(vidx_h.at[pl.ds(s * EMB_CPT, EMB_CPT)], vidx_v)
    col0 = c * HC

    for t in range(EMB_CPT):
        kg = pltpu.async_copy(key_hbm.at[kidx_v.at[t]], kbuf, sem_e)
        vg = pltpu.async_copy(val_hbm.at[vidx_v.at[t]], vbuf, sem_e)
        kg.wait()
        vg.wait()

        def row_body(r, carry):
            for j in range(HC // 16):
                off = col0 + j * 16
                k16 = kbuf[r, pl.ds(off, 16)]
                v16 = vbuf[r, pl.ds(off, 16)]
                hbuf[r, pl.ds(j * 16, 16)] = jnp.maximum(k16 + v16, 0.0)
            return carry

        lax.fori_loop(0, EMB_CHUNK, row_body, 0)
        base = (s * EMB_CPT + t) * EMB_CHUNK
        pltpu.sync_copy(hbuf, h0c_h.at[c, pl.ds(base, EMB_CHUNK)])
    plsc.subcore_barrier()

    yc = h0c_h.at[c]

    def fire_idx(b):
        slot = lax.rem(b, 2)
        pltpu.async_copy(src_h.at[pl.ds(s * CPW + b * BANKC, BANKC)],
                         sidxb.at[slot], sem_i)
        pltpu.async_copy(dst_h.at[pl.ds(s * CPW + b * BANKC, BANKC)],
                         didxb.at[slot], sem_i)

    def drain_idx():
        pltpu.make_async_copy(src_h.at[pl.ds(0, BANKC)], sidxb.at[0],
                              sem_i).wait()
        pltpu.make_async_copy(dst_h.at[pl.ds(0, BANKC)], didxb.at[0],
                              sem_i).wait()

    def fire_gather(slot, k, bank):
        pltpu.async_copy(yc.at[sidxb.at[slot, k]], ebufs.at[bank], sem_g)

    def drain_buf(bank, sem):
        pltpu.make_async_copy(yc.at[pl.ds(0, CHUNK)], ebufs.at[bank],
                              sem).wait()

    fire_idx(0)
    drain_idx()
    fire_gather(0, 0, 0)

    def bank_body(b, carry):
        bslot = lax.rem(b, 2)

        for k in range(BANKC):
            g = b * BANKC + k
            p = lax.rem(g, 2)
            drain_buf(p, sem_g)

            @pl.when(g > 0)
            def _():
                drain_buf(1 - p, sem_s)

            if k == 1:
                @pl.when(b + 1 < NB)
                def _():
                    fire_idx(b + 1)

            if k == BANKC - 1:
                @pl.when(b + 1 < NB)
                def _():
                    drain_idx()
                    fire_gather(1 - bslot, 0, 1 - p)
            else:
                fire_gather(bslot, k + 1, 1 - p)

            pltpu.async_copy(ebufs.at[p], agg_sh.at[didxb.at[bslot, k]],
                             sem_s, add=True)
            pltpu.async_copy(ones_v, deg_sh.at[didxb.at[bslot, k]],
                             sem_d, add=True)
        return carry

    lax.fori_loop(0, NB, bank_body, 0)
    drain_buf(lax.rem(NG - 1, 2), sem_s)
    for _ in range(NB):
        pltpu.make_async_copy(dst_h.at[pl.ds(0, BANKC)], didxb.at[0],
                              sem_d).wait()
    plsc.subcore_barrier()

    pltpu.sync_copy(agg_sh.at[pl.ds(zbase, AGG_PER_TILE)],
                    aggp_h.at[c, pl.ds(zbase, AGG_PER_TILE)])
    pltpu.sync_copy(deg_sh.at[pl.ds(dzbase, DEG_PER_TILE)],
                    degp_h.at[c, pl.ds(dzbase, DEG_PER_TILE)])


@jax.jit
def _sc_l0(key_emb, val_emb, kidx, vidx, src2, dst2, zagg, zdeg, ones):
    kern = pl.kernel(
        _sc_l0_body,
        out_type=(
            jax.ShapeDtypeStruct((NC, N_PAD, HC), jnp.float32),
            jax.ShapeDtypeStruct((NC, AGG_ROWS, HC), jnp.float32),
            jax.ShapeDtypeStruct((NC, DEG_LEN), jnp.float32),
        ),
        mesh=_mesh(),
        scratch_types=[
            pltpu.VMEM((EMB_CPT, EMB_CHUNK), jnp.int32),
            pltpu.VMEM((EMB_CPT, EMB_CHUNK), jnp.int32),
            pltpu.VMEM((EMB_CHUNK, H), jnp.float32),
            pltpu.VMEM((EMB_CHUNK, H), jnp.float32),
            pltpu.VMEM((EMB_CHUNK, HC), jnp.float32),
            pltpu.VMEM((2, BANKC, CHUNK), jnp.int32),
            pltpu.VMEM((2, BANKC, CHUNK), jnp.int32),
            pltpu.VMEM((CHUNK,), jnp.float32),
            pltpu.VMEM((2, CHUNK, HC), jnp.float32),
            pltpu.VMEM_SHARED((AGG_ROWS, HC), jnp.float32),
            pltpu.VMEM_SHARED((DEG_LEN,), jnp.float32),
            pltpu.SemaphoreType.DMA,
            pltpu.SemaphoreType.DMA,
            pltpu.SemaphoreType.DMA,
            pltpu.SemaphoreType.DMA,
            pltpu.SemaphoreType.DMA,
            pltpu.SemaphoreType.DMA,
        ],
        compiler_params=_SC_PARAMS,
    )
    return kern(key_emb, val_emb, kidx, vidx, src2, dst2, zagg, zdeg, ones)


def _sc_spmm_body(y_hbm, src_h, dst_h, zagg_h, aggp_h,
                  sidx_v, didx_v, bufs, agg_sh, sem_g, sem_s):
    c = lax.axis_index("c")
    s = lax.axis_index("s")

    zbase = s * AGG_PER_TILE
    pltpu.sync_copy(zagg_h.at[pl.ds(zbase, AGG_PER_TILE)],
                    agg_sh.at[pl.ds(zbase, AGG_PER_TILE)])

    pltpu.sync_copy(src_h.at[pl.ds(s * CPW, CPW)], sidx_v)
    pltpu.sync_copy(dst_h.at[pl.ds(s * CPW, CPW)], didx_v)

    yc = y_hbm.at[c]

    def fire_gather(g, bank):
        pltpu.async_copy(yc.at[sidx_v.at[g]], bufs.at[bank], sem_g)

    def drain(bank, sem):
        pltpu.make_async_copy(yc.at[pl.ds(0, CHUNK)], bufs.at[bank],
                              sem).wait()

    fire_gather(0, 0)
    plsc.subcore_barrier()

    def body(g, carry):
        p = lax.rem(g, 2)
        drain(p, sem_g)

        @pl.when(g > 0)
        def _():
            drain(1 - p, sem_s)

        @pl.when(g + 1 < NG)
        def _():
            fire_gather(g + 1, 1 - p)

        pltpu.async_copy(bufs.at[p], agg_sh.at[didx_v.at[g]], sem_s, add=True)
        return carry

    lax.fori_loop(0, NG, body, 0)
    drain(lax.rem(NG - 1, 2), sem_s)
    plsc.subcore_barrier()

    pltpu.sync_copy(agg_sh.at[pl.ds(zbase, AGG_PER_TILE)],
                    aggp_h.at[c, pl.ds(zbase, AGG_PER_TILE)])


@jax.jit
def _sc_spmm(y, src2, dst2, zagg):
    kern = pl.kernel(
        _sc_spmm_body,
        out_type=jax.ShapeDtypeStruct((NC, AGG_ROWS, HC), jnp.float32),
        mesh=_mesh(),
        scratch_types=[
            pltpu.VMEM((CPW, CHUNK), jnp.int32),
            pltpu.VMEM((CPW, CHUNK), jnp.int32),
            pltpu.VMEM((2, CHUNK, HC), jnp.float32),
            pltpu.VMEM_SHARED((AGG_ROWS, HC), jnp.float32),
            pltpu.SemaphoreType.DMA,
            pltpu.SemaphoreType.DMA,
        ],
        compiler_params=_SC_PARAMS,
    )
    return kern(y, src2, dst2, zagg)


ROWS_BLK = 1000


def _tc_b_body(h0a_ref, h0b_ref, a0_ref, a1_ref, d_ref,
               ws0_ref, wn0_ref, ws1_ref, wn1_ref, s_ref, y_ref):
    scale = 1.0 / jnp.maximum(d_ref[0], 1.0)
    h0 = jnp.concatenate([h0a_ref[0], h0b_ref[0]], axis=1)
    mean = jnp.concatenate([a0_ref[0], a1_ref[0]], axis=1) * scale
    h1 = (jnp.dot(h0, ws0_ref[...], preferred_element_type=jnp.float32)
          + jnp.dot(mean, wn0_ref[...], preferred_element_type=jnp.float32))
    s_ref[...] = jnp.dot(h1, ws1_ref[...], preferred_element_type=jnp.float32)
    y = jnp.dot(h1, wn1_ref[...], preferred_element_type=jnp.float32)
    y_ref[0] = y[:, :HC]
    y_ref[1] = y[:, HC:]


@jax.jit
def _tc_b(h0c, aggp, degp, ws0, wn0, ws1, wn1):
    grid = (N // ROWS_BLK,)
    blk = pl.BlockSpec((ROWS_BLK, H), lambda i: (i, 0))
    h0a = pl.BlockSpec((1, ROWS_BLK, HC), lambda i: (0, i, 0))
    h0b = pl.BlockSpec((1, ROWS_BLK, HC), lambda i: (1, i, 0))
    a0blk = pl.BlockSpec((1, ROWS_BLK, HC), lambda i: (0, i, 0))
    a1blk = pl.BlockSpec((1, ROWS_BLK, HC), lambda i: (1, i, 0))
    dblk = pl.BlockSpec((1, ROWS_BLK, 1), lambda i: (0, i, 0))
    wblk = pl.BlockSpec((H, H), lambda i: (0, 0))
    yblk = pl.BlockSpec((NC, ROWS_BLK, HC), lambda i: (0, i, 0))
    return pl.pallas_call(
        _tc_b_body,
        grid=grid,
        in_specs=[h0a, h0b, a0blk, a1blk, dblk, wblk, wblk, wblk, wblk],
        out_specs=[blk, yblk],
        out_shape=[
            jax.ShapeDtypeStruct((N, H), jnp.float32),
            jax.ShapeDtypeStruct((NC, N, HC), jnp.float32),
        ],
    )(h0c, h0c, aggp, aggp, degp, ws0, wn0, ws1, wn1)


def _tc_d_body(s1_ref, a0_ref, a1_ref, d_ref, wc_ref, o_ref):
    scale = 1.0 / jnp.maximum(d_ref[0], 1.0)
    agg = jnp.concatenate([a0_ref[0], a1_ref[0]], axis=1)
    h = s1_ref[...] + agg * scale
    o_ref[...] = jnp.dot(h, wc_ref[...], preferred_element_type=jnp.float32)


@jax.jit
def _tc_d(s1, aggp, degp, wc):
    grid = (N // ROWS_BLK,)
    blk = pl.BlockSpec((ROWS_BLK, H), lambda i: (i, 0))
    a0blk = pl.BlockSpec((1, ROWS_BLK, HC), lambda i: (0, i, 0))
    a1blk = pl.BlockSpec((1, ROWS_BLK, HC), lambda i: (1, i, 0))
    dblk = pl.BlockSpec((1, ROWS_BLK, 1), lambda i: (0, i, 0))
    wblk = pl.BlockSpec((H, OUT), lambda i: (0, 0))
    oblk = pl.BlockSpec((ROWS_BLK, OUT), lambda i: (i, 0))
    return pl.pallas_call(
        _tc_d_body,
        grid=grid,
        in_specs=[blk, a0blk, a1blk, dblk, wblk],
        out_specs=oblk,
        out_shape=jax.ShapeDtypeStruct((N, OUT), jnp.float32),
    )(s1, aggp, aggp, degp, wc)


def kernel(feats, edge_index, key_emb, val_emb, W_self0, W_neigh0, W_self1,
           W_neigh1, W_cls):
    kidx = jnp.pad(feats[:, 0], (0, N_PAD - N)).reshape(-1, EMB_CHUNK)
    vidx = jnp.pad(feats[:, 1], (0, N_PAD - N)).reshape(-1, EMB_CHUNK)
    pad_iota = jnp.arange(E_PAD - E, dtype=jnp.int32)
    src2 = jnp.concatenate([edge_index[0], pad_iota * 41 % N]).reshape(
        -1, CHUNK)
    pad_dst = N + pad_iota % 112
    dst2 = jnp.concatenate([edge_index[1], pad_dst]).reshape(-1, CHUNK)
    zdeg = jnp.zeros((DEG_LEN,), jnp.float32)
    zagg = jnp.zeros((AGG_ROWS, HC), jnp.float32)
    ones = jnp.ones((CHUNK,), jnp.float32)

    h0c, aggp0, degp = _sc_l0(key_emb, val_emb, kidx, vidx, src2, dst2,
                              zagg, zdeg, ones)
    degp3 = degp[:1].reshape(1, DEG_LEN, 1)
    s1, y1 = _tc_b(h0c, aggp0, degp3, W_self0, W_neigh0, W_self1, W_neigh1)
    aggp1 = _sc_spmm(y1, src2, dst2, zagg)
    out = _tc_d(s1, aggp1, degp3, W_cls)
    return out


# speedup vs baseline: 2.1087x; 1.0252x over previous
import functools

import jax
import jax.numpy as jnp
from jax import lax
from jax.experimental import pallas as pl
from jax.experimental.pallas import tpu as pltpu
from jax.experimental.pallas import tpu_sc as plsc

N = 10000
E = 320000
H = 128
HC = H // 2
OUT = 64

NC = 2
NS = 16
NW = NC * NS
CHUNK = 256
BANKC = 8
NB = 10
CPW = NB * BANKC
NG = CPW
E_PAD = CPW * NS * CHUNK

EMB_CHUNK = 64
EMB_CPT = -(-N // (NS * EMB_CHUNK))
N_PAD = EMB_CPT * NS * EMB_CHUNK
AGG_ROWS = N + 112
AGG_PER_TILE = AGG_ROWS // NS
DEG_LEN = 10112
DEG_PER_TILE = DEG_LEN // NS


def _mesh():
    return plsc.VectorSubcoreMesh(
        core_axis_name="c", subcore_axis_name="s", num_cores=NC, num_subcores=NS
    )


_SC_PARAMS = pltpu.CompilerParams(use_tc_tiling_on_sc=False)


def _sc_l0_body(key_hbm, val_hbm, kidx_h, vidx_h, src_h, dst_h,
                zagg_h, zdeg_h, ones_h,
                h0c_h, aggp_h, degp_h,
                kidx_v, vidx_v, kbuf, vbuf, hbuf,
                sidxb, didxb, ones_v, ebufs, agg_sh, deg_sh,
                sem_e, sem_h, sem_i, sem_g, sem_s, sem_d):
    c = lax.axis_index("c")
    s = lax.axis_index("s")

    zbase = s * AGG_PER_TILE
    pltpu.sync_copy(zagg_h.at[pl.ds(zbase, AGG_PER_TILE)],
                    agg_sh.at[pl.ds(zbase, AGG_PER_TILE)])
    dzbase = s * DEG_PER_TILE
    pltpu.sync_copy(zdeg_h.at[pl.ds(dzbase, DEG_PER_TILE)],
                    deg_sh.at[pl.ds(dzbase, DEG_PER_TILE)])
    pltpu.sync_copy(ones_h, ones_v)

    pltpu.sync_copy(kidx_h.at[pl.ds(s * EMB_CPT, EMB_CPT)], kidx_v)
    pltpu.sync_copy(vidx_h.at[pl.ds(s * EMB_CPT, EMB_CPT)], vidx_v)
    col0 = c * HC

    def fire_emb(t, slot):
        pltpu.async_copy(key_hbm.at[kidx_v.at[t]], kbuf.at[slot], sem_e)
        pltpu.async_copy(val_hbm.at[vidx_v.at[t]], vbuf.at[slot], sem_e)

    fire_emb(0, 0)
    for t in range(EMB_CPT):
        slot = t % 2
        for _ in range(2):
            pltpu.make_async_copy(key_hbm.at[pl.ds(0, EMB_CHUNK)],
                                  kbuf.at[slot], sem_e).wait()
        if t + 1 < EMB_CPT:
            fire_emb(t + 1, 1 - slot)
        if t >= 2:
            pltpu.make_async_copy(hbuf.at[slot],
                                  h0c_h.at[0, pl.ds(0, EMB_CHUNK)],
                                  sem_h).wait()

        def row_body(r, carry):
            for j in range(HC // 16):
                off = col0 + j * 16
                k16 = kbuf[slot, r, pl.ds(off, 16)]
                v16 = vbuf[slot, r, pl.ds(off, 16)]
                hbuf[slot, r, pl.ds(j * 16, 16)] = jnp.maximum(k16 + v16, 0.0)
            return carry

        lax.fori_loop(0, EMB_CHUNK, row_body, 0)
        base = (s * EMB_CPT + t) * EMB_CHUNK
        pltpu.async_copy(hbuf.at[slot], h0c_h.at[c, pl.ds(base, EMB_CHUNK)],
                         sem_h)
    for slot in range(2):
        pltpu.make_async_copy(hbuf.at[slot],
                              h0c_h.at[0, pl.ds(0, EMB_CHUNK)], sem_h).wait()
    plsc.subcore_barrier()

    yc = h0c_h.at[c]

    def fire_idx(b):
        slot = lax.rem(b, 2)
        pltpu.async_copy(src_h.at[pl.ds(s * CPW + b * BANKC, BANKC)],
                         sidxb.at[slot], sem_i)
        pltpu.async_copy(dst_h.at[pl.ds(s * CPW + b * BANKC, BANKC)],
                         didxb.at[slot], sem_i)

    def drain_idx():
        pltpu.make_async_copy(src_h.at[pl.ds(0, BANKC)], sidxb.at[0],
                              sem_i).wait()
        pltpu.make_async_copy(dst_h.at[pl.ds(0, BANKC)], didxb.at[0],
                              sem_i).wait()

    def fire_gather(slot, k, bank):
        pltpu.async_copy(yc.at[sidxb.at[slot, k]], ebufs.at[bank], sem_g)

    def drain_buf(bank, sem):
        pltpu.make_async_copy(yc.at[pl.ds(0, CHUNK)], ebufs.at[bank],
                              sem).wait()

    fire_idx(0)
    drain_idx()
    fire_gather(0, 0, 0)

    def bank_body(b, carry):
        bslot = lax.rem(b, 2)

        for k in range(BANKC):
            g = b * BANKC + k
            p = lax.rem(g, 2)
            drain_buf(p, sem_g)

            @pl.when(g > 0)
            def _():
                drain_buf(1 - p, sem_s)

            if k == 1:
                @pl.when(b + 1 < NB)
                def _():
                    fire_idx(b + 1)

            if k == BANKC - 1:
                @pl.when(b + 1 < NB)
                def _():
                    drain_idx()
                    fire_gather(1 - bslot, 0, 1 - p)
            else:
                fire_gather(bslot, k + 1, 1 - p)

            pltpu.async_copy(ebufs.at[p], agg_sh.at[didxb.at[bslot, k]],
                             sem_s, add=True)
            pltpu.async_copy(ones_v, deg_sh.at[didxb.at[bslot, k]],
                             sem_d, add=True)
        return carry

    lax.fori_loop(0, NB, bank_body, 0)
    drain_buf(lax.rem(NG - 1, 2), sem_s)
    for _ in range(NB):
        pltpu.make_async_copy(dst_h.at[pl.ds(0, BANKC)], didxb.at[0],
                              sem_d).wait()
    plsc.subcore_barrier()

    pltpu.sync_copy(agg_sh.at[pl.ds(zbase, AGG_PER_TILE)],
                    aggp_h.at[c, pl.ds(zbase, AGG_PER_TILE)])
    pltpu.sync_copy(deg_sh.at[pl.ds(dzbase, DEG_PER_TILE)],
                    degp_h.at[c, pl.ds(dzbase, DEG_PER_TILE)])


@jax.jit
def _sc_l0(key_emb, val_emb, kidx, vidx, src2, dst2, zagg, zdeg, ones):
    kern = pl.kernel(
        _sc_l0_body,
        out_type=(
            jax.ShapeDtypeStruct((NC, N_PAD, HC), jnp.float32),
            jax.ShapeDtypeStruct((NC, AGG_ROWS, HC), jnp.float32),
            jax.ShapeDtypeStruct((NC, DEG_LEN), jnp.float32),
        ),
        mesh=_mesh(),
        scratch_types=[
            pltpu.VMEM((EMB_CPT, EMB_CHUNK), jnp.int32),
            pltpu.VMEM((EMB_CPT, EMB_CHUNK), jnp.int32),
            pltpu.VMEM((2, EMB_CHUNK, H), jnp.float32),
            pltpu.VMEM((2, EMB_CHUNK, H), jnp.float32),
            pltpu.VMEM((2, EMB_CHUNK, HC), jnp.float32),
            pltpu.VMEM((2, BANKC, CHUNK), jnp.int32),
            pltpu.VMEM((2, BANKC, CHUNK), jnp.int32),
            pltpu.VMEM((CHUNK,), jnp.float32),
            pltpu.VMEM((2, CHUNK, HC), jnp.float32),
            pltpu.VMEM_SHARED((AGG_ROWS, HC), jnp.float32),
            pltpu.VMEM_SHARED((DEG_LEN,), jnp.float32),
            pltpu.SemaphoreType.DMA,
            pltpu.SemaphoreType.DMA,
            pltpu.SemaphoreType.DMA,
            pltpu.SemaphoreType.DMA,
            pltpu.SemaphoreType.DMA,
            pltpu.SemaphoreType.DMA,
        ],
        compiler_params=_SC_PARAMS,
    )
    return kern(key_emb, val_emb, kidx, vidx, src2, dst2, zagg, zdeg, ones)


def _sc_spmm_body(y_hbm, src_h, dst_h, zagg_h, aggp_h,
                  sidx_v, didx_v, bufs, agg_sh, sem_g, sem_s):
    c = lax.axis_index("c")
    s = lax.axis_index("s")

    zbase = s * AGG_PER_TILE
    pltpu.sync_copy(zagg_h.at[pl.ds(zbase, AGG_PER_TILE)],
                    agg_sh.at[pl.ds(zbase, AGG_PER_TILE)])

    pltpu.sync_copy(src_h.at[pl.ds(s * CPW, CPW)], sidx_v)
    pltpu.sync_copy(dst_h.at[pl.ds(s * CPW, CPW)], didx_v)

    yc = y_hbm.at[c]

    def fire_gather(g, bank):
        pltpu.async_copy(yc.at[sidx_v.at[g]], bufs.at[bank], sem_g)

    def drain(bank, sem):
        pltpu.make_async_copy(yc.at[pl.ds(0, CHUNK)], bufs.at[bank],
                              sem).wait()

    fire_gather(0, 0)
    plsc.subcore_barrier()

    def body(g, carry):
        p = lax.rem(g, 2)
        drain(p, sem_g)

        @pl.when(g > 0)
        def _():
            drain(1 - p, sem_s)

        @pl.when(g + 1 < NG)
        def _():
            fire_gather(g + 1, 1 - p)

        pltpu.async_copy(bufs.at[p], agg_sh.at[didx_v.at[g]], sem_s, add=True)
        return carry

    lax.fori_loop(0, NG, body, 0)
    drain(lax.rem(NG - 1, 2), sem_s)
    plsc.subcore_barrier()

    pltpu.sync_copy(agg_sh.at[pl.ds(zbase, AGG_PER_TILE)],
                    aggp_h.at[c, pl.ds(zbase, AGG_PER_TILE)])


@jax.jit
def _sc_spmm(y, src2, dst2, zagg):
    kern = pl.kernel(
        _sc_spmm_body,
        out_type=jax.ShapeDtypeStruct((NC, AGG_ROWS, HC), jnp.float32),
        mesh=_mesh(),
        scratch_types=[
            pltpu.VMEM((CPW, CHUNK), jnp.int32),
            pltpu.VMEM((CPW, CHUNK), jnp.int32),
            pltpu.VMEM((2, CHUNK, HC), jnp.float32),
            pltpu.VMEM_SHARED((AGG_ROWS, HC), jnp.float32),
            pltpu.SemaphoreType.DMA,
            pltpu.SemaphoreType.DMA,
        ],
        compiler_params=_SC_PARAMS,
    )
    return kern(y, src2, dst2, zagg)


ROWS_BLK = 1000


def _tc_b_body(h0a_ref, h0b_ref, a0_ref, a1_ref, d_ref,
               ws0_ref, wn0_ref, ws1_ref, wn1_ref, s_ref, y_ref):
    scale = 1.0 / jnp.maximum(d_ref[0], 1.0)
    h0 = jnp.concatenate([h0a_ref[0], h0b_ref[0]], axis=1)
    mean = jnp.concatenate([a0_ref[0], a1_ref[0]], axis=1) * scale
    h1 = (jnp.dot(h0, ws0_ref[...], preferred_element_type=jnp.float32)
          + jnp.dot(mean, wn0_ref[...], preferred_element_type=jnp.float32))
    s_ref[...] = jnp.dot(h1, ws1_ref[...], preferred_element_type=jnp.float32)
    y = jnp.dot(h1, wn1_ref[...], preferred_element_type=jnp.float32)
    y_ref[0] = y[:, :HC]
    y_ref[1] = y[:, HC:]


@jax.jit
def _tc_b(h0c, aggp, degp, ws0, wn0, ws1, wn1):
    grid = (N // ROWS_BLK,)
    blk = pl.BlockSpec((ROWS_BLK, H), lambda i: (i, 0))
    h0a = pl.BlockSpec((1, ROWS_BLK, HC), lambda i: (0, i, 0))
    h0b = pl.BlockSpec((1, ROWS_BLK, HC), lambda i: (1, i, 0))
    a0blk = pl.BlockSpec((1, ROWS_BLK, HC), lambda i: (0, i, 0))
    a1blk = pl.BlockSpec((1, ROWS_BLK, HC), lambda i: (1, i, 0))
    dblk = pl.BlockSpec((1, ROWS_BLK, 1), lambda i: (0, i, 0))
    wblk = pl.BlockSpec((H, H), lambda i: (0, 0))
    yblk = pl.BlockSpec((NC, ROWS_BLK, HC), lambda i: (0, i, 0))
    return pl.pallas_call(
        _tc_b_body,
        grid=grid,
        in_specs=[h0a, h0b, a0blk, a1blk, dblk, wblk, wblk, wblk, wblk],
        out_specs=[blk, yblk],
        out_shape=[
            jax.ShapeDtypeStruct((N, H), jnp.float32),
            jax.ShapeDtypeStruct((NC, N, HC), jnp.float32),
        ],
    )(h0c, h0c, aggp, aggp, degp, ws0, wn0, ws1, wn1)


def _tc_d_body(s1_ref, a0_ref, a1_ref, d_ref, wc_ref, o_ref):
    scale = 1.0 / jnp.maximum(d_ref[0], 1.0)
    agg = jnp.concatenate([a0_ref[0], a1_ref[0]], axis=1)
    h = s1_ref[...] + agg * scale
    o_ref[...] = jnp.dot(h, wc_ref[...], preferred_element_type=jnp.float32)


@jax.jit
def _tc_d(s1, aggp, degp, wc):
    grid = (N // ROWS_BLK,)
    blk = pl.BlockSpec((ROWS_BLK, H), lambda i: (i, 0))
    a0blk = pl.BlockSpec((1, ROWS_BLK, HC), lambda i: (0, i, 0))
    a1blk = pl.BlockSpec((1, ROWS_BLK, HC), lambda i: (1, i, 0))
    dblk = pl.BlockSpec((1, ROWS_BLK, 1), lambda i: (0, i, 0))
    wblk = pl.BlockSpec((H, OUT), lambda i: (0, 0))
    oblk = pl.BlockSpec((ROWS_BLK, OUT), lambda i: (i, 0))
    return pl.pallas_call(
        _tc_d_body,
        grid=grid,
        in_specs=[blk, a0blk, a1blk, dblk, wblk],
        out_specs=oblk,
        out_shape=jax.ShapeDtypeStruct((N, OUT), jnp.float32),
    )(s1, aggp, aggp, degp, wc)


def kernel(feats, edge_index, key_emb, val_emb, W_self0, W_neigh0, W_self1,
           W_neigh1, W_cls):
    kidx = jnp.pad(feats[:, 0], (0, N_PAD - N)).reshape(-1, EMB_CHUNK)
    vidx = jnp.pad(feats[:, 1], (0, N_PAD - N)).reshape(-1, EMB_CHUNK)
    pad_iota = jnp.arange(E_PAD - E, dtype=jnp.int32)
    src2 = jnp.concatenate([edge_index[0], pad_iota * 41 % N]).reshape(
        -1, CHUNK)
    pad_dst = N + pad_iota % 112
    dst2 = jnp.concatenate([edge_index[1], pad_dst]).reshape(-1, CHUNK)
    zdeg = jnp.zeros((DEG_LEN,), jnp.float32)
    zagg = jnp.zeros((AGG_ROWS, HC), jnp.float32)
    ones = jnp.ones((CHUNK,), jnp.float32)

    h0c, aggp0, degp = _sc_l0(key_emb, val_emb, kidx, vidx, src2, dst2,
                              zagg, zdeg, ones)
    degp3 = degp[:1].reshape(1, DEG_LEN, 1)
    s1, y1 = _tc_b(h0c, aggp0, degp3, W_self0, W_neigh0, W_self1, W_neigh1)
    aggp1 = _sc_spmm(y1, src2, dst2, zagg)
    out = _tc_d(s1, aggp1, degp3, W_cls)
    return out


# speedup vs baseline: 2.1207x; 1.0057x over previous
import functools

import jax
import jax.numpy as jnp
from jax import lax
from jax.experimental import pallas as pl
from jax.experimental.pallas import tpu as pltpu
from jax.experimental.pallas import tpu_sc as plsc

N = 10000
E = 320000
H = 128
HC = H // 2
OUT = 64

NC = 2
NS = 16
NW = NC * NS
CHUNK = 256
BANKC = 8
NB = 10
CPW = NB * BANKC
NG = CPW
E_PAD = CPW * NS * CHUNK

EMB_CHUNK = 64
EMB_CPT = -(-N // (NS * EMB_CHUNK))
N_PAD = EMB_CPT * NS * EMB_CHUNK
AGG_ROWS = N + 112
AGG_PER_TILE = AGG_ROWS // NS
DEG_LEN = 10112
DEG_PER_TILE = DEG_LEN // NS


def _mesh():
    return plsc.VectorSubcoreMesh(
        core_axis_name="c", subcore_axis_name="s", num_cores=NC, num_subcores=NS
    )


_SC_PARAMS = pltpu.CompilerParams(use_tc_tiling_on_sc=False)


def _sc_l0_body(key_hbm, val_hbm, kidx_h, vidx_h, src_h, dst_h,
                zagg_h, zdeg_h, ones_h,
                h0c_h, aggp_h, degp_h,
                kidx_v, vidx_v, kbuf, vbuf, hbuf,
                sidxb, didxb, ones_v, ebufs, agg_sh, deg_sh,
                sem_e, sem_h, sem_i, sem_g, sem_s, sem_d):
    c = lax.axis_index("c")
    s = lax.axis_index("s")

    zbase = s * AGG_PER_TILE
    pltpu.sync_copy(zagg_h, agg_sh.at[pl.ds(zbase, AGG_PER_TILE)])
    dzbase = s * DEG_PER_TILE
    pltpu.sync_copy(zdeg_h, deg_sh.at[pl.ds(dzbase, DEG_PER_TILE)])
    pltpu.sync_copy(ones_h, ones_v)

    pltpu.sync_copy(kidx_h.at[pl.ds(s * EMB_CPT, EMB_CPT)], kidx_v)
    pltpu.sync_copy(vidx_h.at[pl.ds(s * EMB_CPT, EMB_CPT)], vidx_v)
    col0 = c * HC

    def fire_emb(t, slot):
        pltpu.async_copy(key_hbm.at[kidx_v.at[t]], kbuf.at[slot], sem_e)
        pltpu.async_copy(val_hbm.at[vidx_v.at[t]], vbuf.at[slot], sem_e)

    fire_emb(0, 0)
    for t in range(EMB_CPT):
        slot = t % 2
        for _ in range(2):
            pltpu.make_async_copy(key_hbm.at[pl.ds(0, EMB_CHUNK)],
                                  kbuf.at[slot], sem_e).wait()
        if t + 1 < EMB_CPT:
            fire_emb(t + 1, 1 - slot)
        if t >= 2:
            pltpu.make_async_copy(hbuf.at[slot],
                                  h0c_h.at[0, pl.ds(0, EMB_CHUNK)],
                                  sem_h).wait()

        def row_body(r, carry):
            for j in range(HC // 16):
                off = col0 + j * 16
                k16 = kbuf[slot, r, pl.ds(off, 16)]
                v16 = vbuf[slot, r, pl.ds(off, 16)]
                hbuf[slot, r, pl.ds(j * 16, 16)] = jnp.maximum(k16 + v16, 0.0)
            return carry

        lax.fori_loop(0, EMB_CHUNK, row_body, 0)
        base = (s * EMB_CPT + t) * EMB_CHUNK
        pltpu.async_copy(hbuf.at[slot], h0c_h.at[c, pl.ds(base, EMB_CHUNK)],
                         sem_h)
    for slot in range(2):
        pltpu.make_async_copy(hbuf.at[slot],
                              h0c_h.at[0, pl.ds(0, EMB_CHUNK)], sem_h).wait()
    plsc.subcore_barrier()

    yc = h0c_h.at[c]

    def fire_idx(b):
        slot = lax.rem(b, 2)
        pltpu.async_copy(src_h.at[pl.ds(s * CPW + b * BANKC, BANKC)],
                         sidxb.at[slot], sem_i)
        pltpu.async_copy(dst_h.at[pl.ds(s * CPW + b * BANKC, BANKC)],
                         didxb.at[slot], sem_i)

    def drain_idx():
        pltpu.make_async_copy(src_h.at[pl.ds(0, BANKC)], sidxb.at[0],
                              sem_i).wait()
        pltpu.make_async_copy(dst_h.at[pl.ds(0, BANKC)], didxb.at[0],
                              sem_i).wait()

    def fire_gather(slot, k, bank):
        pltpu.async_copy(yc.at[sidxb.at[slot, k]], ebufs.at[bank], sem_g)

    def drain_buf(bank, sem):
        pltpu.make_async_copy(yc.at[pl.ds(0, CHUNK)], ebufs.at[bank],
                              sem).wait()

    fire_idx(0)
    drain_idx()
    fire_gather(0, 0, 0)

    def bank_body(b, carry):
        bslot = lax.rem(b, 2)

        for k in range(BANKC):
            g = b * BANKC + k
            p = lax.rem(g, 2)
            drain_buf(p, sem_g)

            @pl.when(g > 0)
            def _():
                drain_buf(1 - p, sem_s)

            if k == 1:
                @pl.when(b + 1 < NB)
                def _():
                    fire_idx(b + 1)

            if k == BANKC - 1:
                @pl.when(b + 1 < NB)
                def _():
                    drain_idx()
                    fire_gather(1 - bslot, 0, 1 - p)
            else:
                fire_gather(bslot, k + 1, 1 - p)

            pltpu.async_copy(ebufs.at[p], agg_sh.at[didxb.at[bslot, k]],
                             sem_s, add=True)
            pltpu.async_copy(ones_v, deg_sh.at[didxb.at[bslot, k]],
                             sem_d, add=True)
        return carry

    lax.fori_loop(0, NB, bank_body, 0)
    drain_buf(lax.rem(NG - 1, 2), sem_s)
    for _ in range(NB):
        pltpu.make_async_copy(dst_h.at[pl.ds(0, BANKC)], didxb.at[0],
                              sem_d).wait()
    plsc.subcore_barrier()

    pltpu.sync_copy(agg_sh.at[pl.ds(zbase, AGG_PER_TILE)],
                    aggp_h.at[c, pl.ds(zbase, AGG_PER_TILE)])
    pltpu.sync_copy(deg_sh.at[pl.ds(dzbase, DEG_PER_TILE)],
                    degp_h.at[c, pl.ds(dzbase, DEG_PER_TILE)])


@jax.jit
def _sc_l0(key_emb, val_emb, kidx, vidx, src2, dst2, zagg, zdeg, ones):
    kern = pl.kernel(
        _sc_l0_body,
        out_type=(
            jax.ShapeDtypeStruct((NC, N_PAD, HC), jnp.float32),
            jax.ShapeDtypeStruct((NC, AGG_ROWS, HC), jnp.float32),
            jax.ShapeDtypeStruct((NC, DEG_LEN), jnp.float32),
        ),
        mesh=_mesh(),
        scratch_types=[
            pltpu.VMEM((EMB_CPT, EMB_CHUNK), jnp.int32),
            pltpu.VMEM((EMB_CPT, EMB_CHUNK), jnp.int32),
            pltpu.VMEM((2, EMB_CHUNK, H), jnp.float32),
            pltpu.VMEM((2, EMB_CHUNK, H), jnp.float32),
            pltpu.VMEM((2, EMB_CHUNK, HC), jnp.float32),
            pltpu.VMEM((2, BANKC, CHUNK), jnp.int32),
            pltpu.VMEM((2, BANKC, CHUNK), jnp.int32),
            pltpu.VMEM((CHUNK,), jnp.float32),
            pltpu.VMEM((2, CHUNK, HC), jnp.float32),
            pltpu.VMEM_SHARED((AGG_ROWS, HC), jnp.float32),
            pltpu.VMEM_SHARED((DEG_LEN,), jnp.float32),
            pltpu.SemaphoreType.DMA,
            pltpu.SemaphoreType.DMA,
            pltpu.SemaphoreType.DMA,
            pltpu.SemaphoreType.DMA,
            pltpu.SemaphoreType.DMA,
            pltpu.SemaphoreType.DMA,
        ],
        compiler_params=_SC_PARAMS,
    )
    return kern(key_emb, val_emb, kidx, vidx, src2, dst2, zagg, zdeg, ones)


def _sc_spmm_body(y_hbm, src_h, dst_h, zagg_h, aggp_h,
                  sidx_v, didx_v, bufs, agg_sh, sem_g, sem_s):
    c = lax.axis_index("c")
    s = lax.axis_index("s")

    zbase = s * AGG_PER_TILE
    pltpu.sync_copy(zagg_h, agg_sh.at[pl.ds(zbase, AGG_PER_TILE)])

    pltpu.sync_copy(src_h.at[pl.ds(s * CPW, CPW)], sidx_v)
    pltpu.sync_copy(dst_h.at[pl.ds(s * CPW, CPW)], didx_v)

    yc = y_hbm.at[c]

    def fire_gather(g, bank):
        pltpu.async_copy(yc.at[sidx_v.at[g]], bufs.at[bank], sem_g)

    def drain(bank, sem):
        pltpu.make_async_copy(yc.at[pl.ds(0, CHUNK)], bufs.at[bank],
                              sem).wait()

    fire_gather(0, 0)
    plsc.subcore_barrier()

    def body(g, carry):
        p = lax.rem(g, 2)
        drain(p, sem_g)

        @pl.when(g > 0)
        def _():
            drain(1 - p, sem_s)

        @pl.when(g + 1 < NG)
        def _():
            fire_gather(g + 1, 1 - p)

        pltpu.async_copy(bufs.at[p], agg_sh.at[didx_v.at[g]], sem_s, add=True)
        return carry

    lax.fori_loop(0, NG, body, 0)
    drain(lax.rem(NG - 1, 2), sem_s)
    plsc.subcore_barrier()

    pltpu.sync_copy(agg_sh.at[pl.ds(zbase, AGG_PER_TILE)],
                    aggp_h.at[c, pl.ds(zbase, AGG_PER_TILE)])


@jax.jit
def _sc_spmm(y, src2, dst2, zagg):
    kern = pl.kernel(
        _sc_spmm_body,
        out_type=jax.ShapeDtypeStruct((NC, AGG_ROWS, HC), jnp.float32),
        mesh=_mesh(),
        scratch_types=[
            pltpu.VMEM((CPW, CHUNK), jnp.int32),
            pltpu.VMEM((CPW, CHUNK), jnp.int32),
            pltpu.VMEM((2, CHUNK, HC), jnp.float32),
            pltpu.VMEM_SHARED((AGG_ROWS, HC), jnp.float32),
            pltpu.SemaphoreType.DMA,
            pltpu.SemaphoreType.DMA,
        ],
        compiler_params=_SC_PARAMS,
    )
    return kern(y, src2, dst2, zagg)


ROWS_BLK = 2000


def _tc_b_body(h0a_ref, h0b_ref, a0_ref, a1_ref, d_ref,
               ws0_ref, wn0_ref, ws1_ref, wn1_ref, s_ref, y_ref):
    scale = 1.0 / jnp.maximum(d_ref[0], 1.0)
    h0 = jnp.concatenate([h0a_ref[0], h0b_ref[0]], axis=1)
    mean = jnp.concatenate([a0_ref[0], a1_ref[0]], axis=1) * scale
    h1 = (jnp.dot(h0, ws0_ref[...], preferred_element_type=jnp.float32)
          + jnp.dot(mean, wn0_ref[...], preferred_element_type=jnp.float32))
    s_ref[...] = jnp.dot(h1, ws1_ref[...], preferred_element_type=jnp.float32)
    y = jnp.dot(h1, wn1_ref[...], preferred_element_type=jnp.float32)
    y_ref[0] = y[:, :HC]
    y_ref[1] = y[:, HC:]


@jax.jit
def _tc_b(h0c, aggp, degp, ws0, wn0, ws1, wn1):
    grid = (N // ROWS_BLK,)
    blk = pl.BlockSpec((ROWS_BLK, H), lambda i: (i, 0))
    h0a = pl.BlockSpec((1, ROWS_BLK, HC), lambda i: (0, i, 0))
    h0b = pl.BlockSpec((1, ROWS_BLK, HC), lambda i: (1, i, 0))
    a0blk = pl.BlockSpec((1, ROWS_BLK, HC), lambda i: (0, i, 0))
    a1blk = pl.BlockSpec((1, ROWS_BLK, HC), lambda i: (1, i, 0))
    dblk = pl.BlockSpec((1, ROWS_BLK, 1), lambda i: (0, i, 0))
    wblk = pl.BlockSpec((H, H), lambda i: (0, 0))
    yblk = pl.BlockSpec((NC, ROWS_BLK, HC), lambda i: (0, i, 0))
    return pl.pallas_call(
        _tc_b_body,
        grid=grid,
        in_specs=[h0a, h0b, a0blk, a1blk, dblk, wblk, wblk, wblk, wblk],
        out_specs=[blk, yblk],
        out_shape=[
            jax.ShapeDtypeStruct((N, H), jnp.float32),
            jax.ShapeDtypeStruct((NC, N, HC), jnp.float32),
        ],
    )(h0c, h0c, aggp, aggp, degp, ws0, wn0, ws1, wn1)


def _tc_d_body(s1_ref, a0_ref, a1_ref, d_ref, wc_ref, o_ref):
    scale = 1.0 / jnp.maximum(d_ref[0], 1.0)
    agg = jnp.concatenate([a0_ref[0], a1_ref[0]], axis=1)
    h = s1_ref[...] + agg * scale
    o_ref[...] = jnp.dot(h, wc_ref[...], preferred_element_type=jnp.float32)


@jax.jit
def _tc_d(s1, aggp, degp, wc):
    grid = (N // ROWS_BLK,)
    blk = pl.BlockSpec((ROWS_BLK, H), lambda i: (i, 0))
    a0blk = pl.BlockSpec((1, ROWS_BLK, HC), lambda i: (0, i, 0))
    a1blk = pl.BlockSpec((1, ROWS_BLK, HC), lambda i: (1, i, 0))
    dblk = pl.BlockSpec((1, ROWS_BLK, 1), lambda i: (0, i, 0))
    wblk = pl.BlockSpec((H, OUT), lambda i: (0, 0))
    oblk = pl.BlockSpec((ROWS_BLK, OUT), lambda i: (i, 0))
    return pl.pallas_call(
        _tc_d_body,
        grid=grid,
        in_specs=[blk, a0blk, a1blk, dblk, wblk],
        out_specs=oblk,
        out_shape=jax.ShapeDtypeStruct((N, OUT), jnp.float32),
    )(s1, aggp, aggp, degp, wc)


def kernel(feats, edge_index, key_emb, val_emb, W_self0, W_neigh0, W_self1,
           W_neigh1, W_cls):
    kidx = jnp.pad(feats[:, 0], (0, N_PAD - N)).reshape(-1, EMB_CHUNK)
    vidx = jnp.pad(feats[:, 1], (0, N_PAD - N)).reshape(-1, EMB_CHUNK)
    pad_iota = jnp.arange(E_PAD - E, dtype=jnp.int32)
    src2 = jnp.concatenate([edge_index[0], pad_iota * 41 % N]).reshape(
        -1, CHUNK)
    pad_dst = N + pad_iota % 112
    dst2 = jnp.concatenate([edge_index[1], pad_dst]).reshape(-1, CHUNK)
    zdeg = jnp.zeros((DEG_PER_TILE,), jnp.float32)
    zagg = jnp.zeros((AGG_PER_TILE, HC), jnp.float32)
    ones = jnp.ones((CHUNK,), jnp.float32)

    h0c, aggp0, degp = _sc_l0(key_emb, val_emb, kidx, vidx, src2, dst2,
                              zagg, zdeg, ones)
    degp3 = degp[:1].reshape(1, DEG_LEN, 1)
    s1, y1 = _tc_b(h0c, aggp0, degp3, W_self0, W_neigh0, W_self1, W_neigh1)
    aggp1 = _sc_spmm(y1, src2, dst2, zagg)
    out = _tc_d(s1, aggp1, degp3, W_cls)
    return out
